# baseline (speedup 1.0000x reference)
import functools

import jax
import jax.numpy as jnp
from jax import lax
from jax.experimental import pallas as pl
from jax.experimental.pallas import tpu as pltpu
from jax.experimental.pallas import tpu_sc as plsc

N_NODES = 100000
OUT_DIM = 128
NE = 150000
PADN = 150528
ROWS_PER_ETYPE = PADN // OUT_DIM
NTILES = 32
EPT = PADN // 8
CH = 128
NCHUNK = EPT // CH
TOT = 4 * PADN
REG = 0.01


def _sc_scores(table, src, dst, wmat):
    mesh = plsc.VectorSubcoreMesh(core_axis_name="c", subcore_axis_name="s")

    @functools.partial(
        pl.kernel,
        mesh=mesh,
        out_type=jax.ShapeDtypeStruct((TOT,), jnp.float32),
        compiler_params=pltpu.CompilerParams(needs_layout_passes=False),
        scratch_types=[
            pltpu.VMEM((CH,), jnp.int32),
            pltpu.VMEM((CH,), jnp.int32),
            pltpu.VMEM((CH, OUT_DIM), jnp.float32),
            pltpu.VMEM((CH, OUT_DIM), jnp.float32),
            pltpu.VMEM((CH,), jnp.float32),
            pltpu.VMEM((OUT_DIM,), jnp.float32),
            pltpu.SemaphoreType.DMA,
            pltpu.SemaphoreType.DMA,
        ],
    )
    def k(table_hbm, src_hbm, dst_hbm, wmat_hbm, out_hbm,
          sidx, didx, srows, orows, scores, wrow, sem_s, sem_o):
        wid = lax.axis_index("s") * 2 + lax.axis_index("c")
        etype = wid // 8
        base = wid * EPT
        pltpu.sync_copy(wmat_hbm.at[etype], wrow)
        wv = [wrow[pl.ds(kk * 16, 16)] for kk in range(8)]
        last_lane = lax.iota(jnp.int32, 16) == 15

        def chunk_body(g, carry):
            off = base + g * CH
            pltpu.sync_copy(src_hbm.at[pl.ds(off, CH)], sidx)
            pltpu.sync_copy(dst_hbm.at[pl.ds(off, CH)], didx)
            cs = pltpu.async_copy(table_hbm.at[sidx], srows, sem_s)
            co = pltpu.async_copy(table_hbm.at[didx], orows, sem_o)
            cs.wait()
            co.wait()

            def edge_body(e, c2):
                acc = (srows[e, pl.ds(0, 16)] * wv[0]) * orows[e, pl.ds(0, 16)]
                for kk in range(1, 8):
                    acc = acc + (srows[e, pl.ds(kk * 16, 16)] * wv[kk]) \
                        * orows[e, pl.ds(kk * 16, 16)]
                tot = jnp.full((16,), jnp.sum(acc))
                eidx = jnp.full((16,), e, jnp.int32)
                plsc.store_scatter(scores, [eidx], tot, mask=last_lane)
                return c2

            lax.fori_loop(0, CH, edge_body, 0)
            pltpu.sync_copy(scores, out_hbm.at[pl.ds(off, CH)])
            return carry

        lax.fori_loop(0, NCHUNK, chunk_body, 0)

    return k(table, src, dst, wmat)


def _tc_loss(scores4, labels4, embed, wmat):
    emb_blk = 4000
    n_blk = N_NODES // emb_blk

    def body(scores_ref, labels_ref, wmat_ref, embed_ref, out_ref):
        i = pl.program_id(0)

        @pl.when(i == 0)
        def _init():
            x = scores_ref[...]
            y = labels_ref[...]
            row = lax.broadcasted_iota(jnp.int32, x.shape, 0)
            col = lax.broadcasted_iota(jnp.int32, x.shape, 1)
            rin = row % ROWS_PER_ETYPE
            valid = (rin * OUT_DIM + col) < NE
            bce = jnp.maximum(x, 0.0) - x * y + jnp.log1p(jnp.exp(-jnp.abs(x)))
            bce = jnp.where(valid, bce, 0.0)
            w = wmat_ref[...]
            out_ref[0, 0] = jnp.sum(bce) / NE + REG * (jnp.sum(w * w) / OUT_DIM)

        blk = embed_ref[...]
        out_ref[0, 0] += REG * jnp.sum(blk * blk) / (N_NODES * OUT_DIM)

    out = pl.pallas_call(
        body,
        grid=(n_blk,),
        in_specs=[
            pl.BlockSpec((4 * ROWS_PER_ETYPE, OUT_DIM), lambda i: (0, 0)),
            pl.BlockSpec((4 * ROWS_PER_ETYPE, OUT_DIM), lambda i: (0, 0)),
            pl.BlockSpec((4, OUT_DIM), lambda i: (0, 0)),
            pl.BlockSpec((emb_blk, OUT_DIM), lambda i: (i, 0)),
        ],
        out_specs=pl.BlockSpec(memory_space=pltpu.SMEM),
        out_shape=jax.ShapeDtypeStruct((1, 1), jnp.float32),
    )(scores4, labels4, wmat, embed)
    return out[0, 0]


def kernel(embed_0,
           edges_rel0, edges_rel1, edges_rel2, edges_rel3,
           labels_rel0, labels_rel1, labels_rel2, labels_rel3,
           w_rel0, w_rel1, w_rel2, w_rel3):
    edges = [edges_rel0, edges_rel1, edges_rel2, edges_rel3]
    labels = [labels_rel0, labels_rel1, labels_rel2, labels_rel3]
    pad = PADN - NE
    src = jnp.concatenate([jnp.pad(ed[:, 0], (0, pad)) for ed in edges])
    dst = jnp.concatenate([jnp.pad(ed[:, 1], (0, pad)) for ed in edges])
    lab = jnp.concatenate([jnp.pad(lb, (0, pad)) for lb in labels])
    wmat = jnp.stack([w_rel0, w_rel1, w_rel2, w_rel3])

    scores = _sc_scores(embed_0, src, dst, wmat)
    return _tc_loss(scores.reshape(4 * ROWS_PER_ETYPE, OUT_DIM),
                    lab.reshape(4 * ROWS_PER_ETYPE, OUT_DIM),
                    embed_0, wmat)
